# baseline (speedup 1.0000x reference)
import dataclasses
import functools

import jax
import jax.numpy as jnp
from jax import lax
from jax.experimental import pallas as pl
from jax.experimental.pallas import tpu as pltpu
from jax.experimental.pallas import tpu_sc as plsc

N_NODES = 10000
N_EDGES = 320000
D = 128

NC = 2
NS = 16
LANES = 16

E_PER_SUB = N_EDGES // (NC * NS)
CHUNK = 80
N_CHUNKS = E_PER_SUB // CHUNK
NROW = 2
NSLOT = 4
N_BODY = N_CHUNKS // NSLOT
N_EPI = N_CHUNKS - N_BODY * NSLOT
ROWS_PER_SUB = 624
TAIL_ROWS = N_NODES - NS * ROWS_PER_SUB


def _sc_aggregate(x, src, dst, w):
    mesh = plsc.VectorSubcoreMesh(core_axis_name="c", subcore_axis_name="s")
    cp = pltpu.CompilerParams()
    if "needs_layout_passes" in pltpu.CompilerParams.__dataclass_fields__:
        cp = dataclasses.replace(cp, needs_layout_passes=False)

    @functools.partial(
        pl.kernel,
        out_type=jax.ShapeDtypeStruct((NC, N_NODES, D), jnp.float32),
        mesh=mesh,
        compiler_params=cp,
        scratch_types=[
            *[pltpu.VMEM((CHUNK,), jnp.int32) for _ in range(NSLOT)],
            *[pltpu.VMEM((CHUNK,), jnp.int32) for _ in range(NSLOT)],
            *[pltpu.VMEM((CHUNK,), jnp.float32) for _ in range(NSLOT)],
            *[pltpu.VMEM((CHUNK, D), jnp.float32) for _ in range(NROW)],
            *[pltpu.SemaphoreType.DMA for _ in range(3 * NSLOT + 2 * NROW)],
            pltpu.VMEM_SHARED((N_NODES, D), jnp.float32),
        ],
    )
    def sc_kernel(x_hbm, src_hbm, dst_hbm, w_hbm, out_hbm, *bufs_sems_acc):
        src_idx_vs = bufs_sems_acc[:NSLOT]
        dst_idx_vs = bufs_sems_acc[NSLOT:2 * NSLOT]
        w_vs = bufs_sems_acc[2 * NSLOT:3 * NSLOT]
        rows_vs = bufs_sems_acc[3 * NSLOT:3 * NSLOT + NROW]
        sems = bufs_sems_acc[3 * NSLOT + NROW:3 * NSLOT + NROW + 3 * NSLOT
                             + 2 * NROW]
        srcsems = sems[:NSLOT]
        dstsems = sems[NSLOT:2 * NSLOT]
        wsems = sems[2 * NSLOT:3 * NSLOT]
        gsems = sems[3 * NSLOT:3 * NSLOT + NROW]
        ssems = sems[3 * NSLOT + NROW:3 * NSLOT + 2 * NROW]
        acc_sh = bufs_sems_acc[-1]
        c = lax.axis_index("c")
        s = lax.axis_index("s")
        ebase = (c * NS + s) * E_PER_SUB

        def stage_chunk(u, eoff):
            pltpu.async_copy(src_hbm.at[pl.ds(ebase + eoff, CHUNK)],
                             src_idx_vs[u], srcsems[u])
            pltpu.async_copy(dst_hbm.at[pl.ds(ebase + eoff, CHUNK)],
                             dst_idx_vs[u], dstsems[u])
            pltpu.async_copy(w_hbm.at[pl.ds(ebase + eoff, CHUNK)],
                             w_vs[u], wsems[u])

        def wait_src(u):
            pltpu.make_async_copy(src_hbm.at[pl.ds(ebase, CHUNK)],
                                  src_idx_vs[u], srcsems[u]).wait()

        def wait_dst(u):
            pltpu.make_async_copy(dst_hbm.at[pl.ds(ebase, CHUNK)],
                                  dst_idx_vs[u], dstsems[u]).wait()

        def wait_w(u):
            pltpu.make_async_copy(w_hbm.at[pl.ds(ebase, CHUNK)],
                                  w_vs[u], wsems[u]).wait()

        def fire_gather(u, r):
            pltpu.async_copy(x_hbm.at[src_idx_vs[u]], rows_vs[r], gsems[r])

        def wait_gather(u, r):
            pltpu.make_async_copy(x_hbm.at[src_idx_vs[u]],
                                  rows_vs[r], gsems[r]).wait()

        def fire_scatter(u, r):
            pltpu.async_copy(rows_vs[r], acc_sh.at[dst_idx_vs[u]], ssems[r],
                             add=True)

        def wait_scatter(u, r):
            pltpu.make_async_copy(rows_vs[r],
                                  acc_sh.at[dst_idx_vs[u]], ssems[r]).wait()

        def scale_rows(u, r):
            @plsc.parallel_loop(0, CHUNK, unroll=4)
            def _row(i):
                wb = plsc.load_gather(
                    w_vs[u], [jnp.full((LANES,), i, jnp.int32)])
                for j in range(D // LANES):
                    sl = pl.ds(j * LANES, LANES)
                    rows_vs[r][i, sl] = rows_vs[r][i, sl] * wb

        for u in range(NSLOT):
            stage_chunk(u, u * CHUNK)

        rbase = s * ROWS_PER_SUB

        @pl.loop(0, CHUNK)
        def _zrow(i):
            for j in range(D // LANES):
                rows_vs[0][i, pl.ds(j * LANES, LANES)] = jnp.zeros(
                    (LANES,), jnp.float32)

        zhs = []
        for k in range(ROWS_PER_SUB // CHUNK):
            zhs.append(pltpu.async_copy(
                rows_vs[0], acc_sh.at[pl.ds(rbase + k * CHUNK, CHUNK)],
                ssems[0]))
        _ZREM = ROWS_PER_SUB % CHUNK
        zhs.append(pltpu.async_copy(
            rows_vs[0].at[pl.ds(0, _ZREM)],
            acc_sh.at[pl.ds(rbase + (ROWS_PER_SUB // CHUNK) * CHUNK, _ZREM)],
            ssems[0]))
        for zh in zhs:
            zh.wait()

        @pl.when(s == NS - 1)
        def _init_tail():
            tbase = NS * ROWS_PER_SUB
            pltpu.sync_copy(rows_vs[0].at[pl.ds(0, TAIL_ROWS)],
                            acc_sh.at[pl.ds(tbase, TAIL_ROWS)])

        plsc.subcore_barrier()

        for r in range(NROW):
            wait_src(r)
            fire_gather(r, r)

        @pl.loop(0, N_BODY)
        def _body(g):
            cbase = g * NSLOT
            for k in range(NSLOT):
                u, r = k, k % NROW
                cc = cbase + k
                wait_gather(u, r)
                wait_w(u)
                scale_rows(u, r)
                wait_dst(u)
                fire_scatter(u, r)
                wait_scatter(u, r)

                @pl.when(cc + NSLOT < N_CHUNKS)
                def _restage():
                    stage_chunk(u, (cc + NSLOT) * CHUNK)

                u4 = (k + NROW) % NSLOT

                @pl.when(cc + NROW < N_CHUNKS)
                def _next_gather():
                    wait_src(u4)
                    fire_gather(u4, r)

        for e in range(N_EPI):
            cc = N_BODY * NSLOT + e
            u, r = cc % NSLOT, cc % NROW
            wait_gather(u, r)
            wait_w(u)
            scale_rows(u, r)
            wait_dst(u)
            fire_scatter(u, r)
            wait_scatter(u, r)
            if cc + NROW < N_CHUNKS:
                u4 = (cc + NROW) % NSLOT
                wait_src(u4)
                fire_gather(u4, r)

        plsc.subcore_barrier()
        pltpu.sync_copy(acc_sh.at[pl.ds(rbase, ROWS_PER_SUB)],
                        out_hbm.at[c].at[pl.ds(rbase, ROWS_PER_SUB)])

        @pl.when(s == NS - 1)
        def _out_tail():
            tbase = NS * ROWS_PER_SUB
            pltpu.sync_copy(acc_sh.at[pl.ds(tbase, TAIL_ROWS)],
                            out_hbm.at[c].at[pl.ds(tbase, TAIL_ROWS)])

    return sc_kernel(x, src, dst, w)


def _tc_mlp_body(agg_ref, x_ref, w1_ref, b1_ref, g_ref, bt_ref, w2_ref,
                 b2_ref, y_ref):
    dn = (((1,), (1,)), ((), ()))
    out = agg_ref[0] + agg_ref[1] + x_ref[...]
    h = lax.dot_general(out, w1_ref[...], dn,
                        preferred_element_type=jnp.float32)
    h = h + b1_ref[...][None, :]
    mu = jnp.mean(h, axis=0, keepdims=True)
    d = h - mu
    var = jnp.mean(d * d, axis=0, keepdims=True)
    hn = d * lax.rsqrt(var + 1e-5) * g_ref[...][None, :] + bt_ref[...][None, :]
    hr = jnp.maximum(hn, 0.0)
    y = lax.dot_general(hr, w2_ref[...], dn,
                        preferred_element_type=jnp.float32)
    y_ref[...] = y + b2_ref[...][None, :]


def kernel(x, edge_index, edge_attr, edge_weight, W1, b1, gamma, beta, W2, b2):
    del edge_attr
    src = edge_index[0].astype(jnp.int32)
    dst = edge_index[1].astype(jnp.int32)
    w = edge_weight.astype(jnp.float32)

    agg = _sc_aggregate(x, src, dst, w)

    return pl.pallas_call(
        _tc_mlp_body,
        out_shape=jax.ShapeDtypeStruct((N_NODES, D), jnp.float32),
    )(agg, x, W1, b1, gamma, beta, W2, b2)


# speedup vs baseline: 1.1089x; 1.1089x over previous
import dataclasses
import functools

import jax
import jax.numpy as jnp
from jax import lax
from jax.experimental import pallas as pl
from jax.experimental.pallas import tpu as pltpu
from jax.experimental.pallas import tpu_sc as plsc

N_NODES = 10000
N_EDGES = 320000
D = 128

NC = 2
NS = 16
LANES = 16

E_PER_SUB = N_EDGES // (NC * NS)
CHUNK = 80
N_CHUNKS = E_PER_SUB // CHUNK
NROW = 3
NSLOT = 6
N_BODY = N_CHUNKS // NSLOT
N_EPI = N_CHUNKS - N_BODY * NSLOT
ROWS_PER_SUB = 624
TAIL_ROWS = N_NODES - NS * ROWS_PER_SUB


def _sc_aggregate(x, src, dst, w):
    mesh = plsc.VectorSubcoreMesh(core_axis_name="c", subcore_axis_name="s")
    cp = pltpu.CompilerParams()
    if "needs_layout_passes" in pltpu.CompilerParams.__dataclass_fields__:
        cp = dataclasses.replace(cp, needs_layout_passes=False)

    @functools.partial(
        pl.kernel,
        out_type=jax.ShapeDtypeStruct((NC, N_NODES, D), jnp.float32),
        mesh=mesh,
        compiler_params=cp,
        scratch_types=[
            *[pltpu.VMEM((CHUNK,), jnp.int32) for _ in range(NSLOT)],
            *[pltpu.VMEM((CHUNK,), jnp.int32) for _ in range(NSLOT)],
            *[pltpu.VMEM((CHUNK,), jnp.float32) for _ in range(NSLOT)],
            *[pltpu.VMEM((CHUNK, D), jnp.float32) for _ in range(NROW)],
            *[pltpu.SemaphoreType.DMA for _ in range(3 * NSLOT + 2 * NROW)],
            pltpu.VMEM_SHARED((N_NODES, D), jnp.float32),
        ],
    )
    def sc_kernel(x_hbm, src_hbm, dst_hbm, w_hbm, out_hbm, *bufs_sems_acc):
        src_idx_vs = bufs_sems_acc[:NSLOT]
        dst_idx_vs = bufs_sems_acc[NSLOT:2 * NSLOT]
        w_vs = bufs_sems_acc[2 * NSLOT:3 * NSLOT]
        rows_vs = bufs_sems_acc[3 * NSLOT:3 * NSLOT + NROW]
        sems = bufs_sems_acc[3 * NSLOT + NROW:3 * NSLOT + NROW + 3 * NSLOT
                             + 2 * NROW]
        srcsems = sems[:NSLOT]
        dstsems = sems[NSLOT:2 * NSLOT]
        wsems = sems[2 * NSLOT:3 * NSLOT]
        gsems = sems[3 * NSLOT:3 * NSLOT + NROW]
        ssems = sems[3 * NSLOT + NROW:3 * NSLOT + 2 * NROW]
        acc_sh = bufs_sems_acc[-1]
        c = lax.axis_index("c")
        s = lax.axis_index("s")
        ebase = (c * NS + s) * E_PER_SUB

        def stage_chunk(u, eoff):
            pltpu.async_copy(src_hbm.at[pl.ds(ebase + eoff, CHUNK)],
                             src_idx_vs[u], srcsems[u])
            pltpu.async_copy(dst_hbm.at[pl.ds(ebase + eoff, CHUNK)],
                             dst_idx_vs[u], dstsems[u])
            pltpu.async_copy(w_hbm.at[pl.ds(ebase + eoff, CHUNK)],
                             w_vs[u], wsems[u])

        def wait_src(u):
            pltpu.make_async_copy(src_hbm.at[pl.ds(ebase, CHUNK)],
                                  src_idx_vs[u], srcsems[u]).wait()

        def wait_dst(u):
            pltpu.make_async_copy(dst_hbm.at[pl.ds(ebase, CHUNK)],
                                  dst_idx_vs[u], dstsems[u]).wait()

        def wait_w(u):
            pltpu.make_async_copy(w_hbm.at[pl.ds(ebase, CHUNK)],
                                  w_vs[u], wsems[u]).wait()

        def fire_gather(u, r):
            pltpu.async_copy(x_hbm.at[src_idx_vs[u]], rows_vs[r], gsems[r])

        def wait_gather(u, r):
            pltpu.make_async_copy(x_hbm.at[src_idx_vs[u]],
                                  rows_vs[r], gsems[r]).wait()

        def fire_scatter(u, r):
            pltpu.async_copy(rows_vs[r], acc_sh.at[dst_idx_vs[u]], ssems[r],
                             add=True)

        def wait_scatter(u, r):
            pltpu.make_async_copy(rows_vs[r],
                                  acc_sh.at[dst_idx_vs[u]], ssems[r]).wait()

        def scale_rows(u, r):
            @plsc.parallel_loop(0, CHUNK, unroll=4)
            def _row(i):
                wb = plsc.load_gather(
                    w_vs[u], [jnp.full((LANES,), i, jnp.int32)])
                for j in range(D // LANES):
                    sl = pl.ds(j * LANES, LANES)
                    rows_vs[r][i, sl] = rows_vs[r][i, sl] * wb

        for u in range(NSLOT):
            stage_chunk(u, u * CHUNK)

        rbase = s * ROWS_PER_SUB

        @pl.loop(0, CHUNK)
        def _zrow(i):
            for j in range(D // LANES):
                rows_vs[0][i, pl.ds(j * LANES, LANES)] = jnp.zeros(
                    (LANES,), jnp.float32)

        zhs = []
        for k in range(ROWS_PER_SUB // CHUNK):
            zhs.append(pltpu.async_copy(
                rows_vs[0], acc_sh.at[pl.ds(rbase + k * CHUNK, CHUNK)],
                ssems[0]))
        _ZREM = ROWS_PER_SUB % CHUNK
        zhs.append(pltpu.async_copy(
            rows_vs[0].at[pl.ds(0, _ZREM)],
            acc_sh.at[pl.ds(rbase + (ROWS_PER_SUB // CHUNK) * CHUNK, _ZREM)],
            ssems[0]))
        for zh in zhs:
            zh.wait()

        @pl.when(s == NS - 1)
        def _init_tail():
            tbase = NS * ROWS_PER_SUB
            pltpu.sync_copy(rows_vs[0].at[pl.ds(0, TAIL_ROWS)],
                            acc_sh.at[pl.ds(tbase, TAIL_ROWS)])

        plsc.subcore_barrier()

        for rr in range(NROW - 1):
            wait_src(rr)
            fire_gather(rr, rr)

        @pl.loop(0, N_BODY)
        def _body(g):
            cbase = g * NSLOT
            for k in range(NSLOT):
                u, r = k, k % NROW
                u1, r1 = (k - 1) % NSLOT, (k - 1) % NROW
                u2 = (k + NROW - 1) % NSLOT
                cc = cbase + k
                wait_gather(u, r)
                wait_w(u)
                scale_rows(u, r)
                wait_dst(u)
                fire_scatter(u, r)

                @pl.when(cc >= 1)
                def _prev_scatter_wait():
                    wait_scatter(u1, r1)

                @pl.when((cc >= 1) & (cc + NSLOT - 1 < N_CHUNKS))
                def _restage():
                    stage_chunk(u1, (cc + NSLOT - 1) * CHUNK)

                @pl.when(cc + NROW - 1 < N_CHUNKS)
                def _next_gather():
                    wait_src(u2)
                    fire_gather(u2, r1)

        for e in range(N_EPI):
            cc = N_BODY * NSLOT + e
            u, r = cc % NSLOT, cc % NROW
            u1, r1 = (cc - 1) % NSLOT, (cc - 1) % NROW
            wait_gather(u, r)
            wait_w(u)
            scale_rows(u, r)
            wait_dst(u)
            fire_scatter(u, r)
            wait_scatter(u1, r1)
            if cc + NROW - 1 < N_CHUNKS:
                u2 = (cc + NROW - 1) % NSLOT
                wait_src(u2)
                fire_gather(u2, r1)
        wait_scatter((N_CHUNKS - 1) % NSLOT, (N_CHUNKS - 1) % NROW)

        plsc.subcore_barrier()
        pltpu.sync_copy(acc_sh.at[pl.ds(rbase, ROWS_PER_SUB)],
                        out_hbm.at[c].at[pl.ds(rbase, ROWS_PER_SUB)])

        @pl.when(s == NS - 1)
        def _out_tail():
            tbase = NS * ROWS_PER_SUB
            pltpu.sync_copy(acc_sh.at[pl.ds(tbase, TAIL_ROWS)],
                            out_hbm.at[c].at[pl.ds(tbase, TAIL_ROWS)])

    return sc_kernel(x, src, dst, w)


def _tc_mlp_body(agg_ref, x_ref, w1_ref, b1_ref, g_ref, bt_ref, w2_ref,
                 b2_ref, y_ref):
    dn = (((1,), (1,)), ((), ()))
    out = agg_ref[0] + agg_ref[1] + x_ref[...]
    h = lax.dot_general(out, w1_ref[...], dn,
                        preferred_element_type=jnp.float32)
    h = h + b1_ref[...][None, :]
    mu = jnp.mean(h, axis=0, keepdims=True)
    d = h - mu
    var = jnp.mean(d * d, axis=0, keepdims=True)
    hn = d * lax.rsqrt(var + 1e-5) * g_ref[...][None, :] + bt_ref[...][None, :]
    hr = jnp.maximum(hn, 0.0)
    y = lax.dot_general(hr, w2_ref[...], dn,
                        preferred_element_type=jnp.float32)
    y_ref[...] = y + b2_ref[...][None, :]


def kernel(x, edge_index, edge_attr, edge_weight, W1, b1, gamma, beta, W2, b2):
    del edge_attr
    src = edge_index[0].astype(jnp.int32)
    dst = edge_index[1].astype(jnp.int32)
    w = edge_weight.astype(jnp.float32)

    agg = _sc_aggregate(x, src, dst, w)

    return pl.pallas_call(
        _tc_mlp_body,
        out_shape=jax.ShapeDtypeStruct((N_NODES, D), jnp.float32),
    )(agg, x, W1, b1, gamma, beta, W2, b2)
